# baseline (speedup 1.0000x reference)
import dataclasses
import functools

import jax
import jax.numpy as jnp
from jax import lax
from jax.experimental import pallas as pl
from jax.experimental.pallas import tpu as pltpu
from jax.experimental.pallas import tpu_sc as plsc

N = 10000
E = 320000
D = 128

NC = 2
NS = 16
NW = NC * NS
CH = 128
NF = 80
EPT = NF * CH
EP = NW * EPT
NB = 2
NP = 10240
RPT = NP // NS
ZR = 32

_mesh = plsc.VectorSubcoreMesh(core_axis_name="c", subcore_axis_name="s")

_cp = pltpu.CompilerParams()
if "needs_layout_passes" in pltpu.CompilerParams.__dataclass_fields__:
    _cp = dataclasses.replace(_cp, needs_layout_passes=False)


@functools.partial(
    pl.kernel,
    out_type=jax.ShapeDtypeStruct((NC * NP,), jnp.float32),
    mesh=_mesh,
    scratch_types=[
        pltpu.VMEM((NP,), jnp.float32),
        pltpu.VMEM((EPT,), jnp.int32),
        pltpu.VMEM_SHARED((NS, NP), jnp.float32),
        pltpu.VMEM((RPT,), jnp.float32),
        pltpu.VMEM((RPT,), jnp.float32),
    ],
    compiler_params=_cp,
)
def _deg_kernel(dst_hbm, out_hbm, hist_v, idx_v, shared, tmp_v, acc_v):
    c = lax.axis_index("c")
    s = lax.axis_index("s")
    wid = c * NS + s
    base = wid * EPT
    pltpu.sync_copy(dst_hbm.at[pl.ds(base, EPT)], idx_v)

    zero16 = jnp.zeros((16,), jnp.float32)
    ones16 = jnp.ones((16,), jnp.float32)

    @pl.loop(0, NP // 16)
    def _(i):
        hist_v[pl.ds(i * 16, 16)] = zero16

    @pl.loop(0, EPT // 16)
    def _(j):
        idx16 = idx_v[pl.ds(j * 16, 16)]
        plsc.addupdate_scatter(hist_v, [idx16], ones16)

    pltpu.sync_copy(hist_v, shared.at[s])
    plsc.subcore_barrier()

    pltpu.sync_copy(shared.at[0, pl.ds(s * RPT, RPT)], acc_v)

    @pl.loop(1, NS)
    def _(r):
        pltpu.sync_copy(shared.at[r, pl.ds(s * RPT, RPT)], tmp_v)

        @pl.loop(0, RPT // 16)
        def _(i):
            sl = pl.ds(i * 16, 16)
            acc_v[sl] = acc_v[sl] + tmp_v[sl]

    pltpu.sync_copy(acc_v, out_hbm.at[pl.ds(c * NP + s * RPT, RPT)])


@functools.partial(
    pl.kernel,
    out_type=jax.ShapeDtypeStruct((NC * NP, D), jnp.float32),
    mesh=_mesh,
    scratch_types=[
        pltpu.VMEM_SHARED((NP, D), jnp.float32),
        pltpu.VMEM((CH,), jnp.int32),
        pltpu.VMEM((CH,), jnp.int32),
        pltpu.VMEM((CH,), jnp.int32),
        pltpu.VMEM((CH,), jnp.int32),
        pltpu.VMEM((CH, D), jnp.float32),
        pltpu.VMEM((CH, D), jnp.float32),
        pltpu.VMEM((ZR, D), jnp.float32),
        pltpu.SemaphoreType.DMA,
        pltpu.SemaphoreType.DMA,
    ],
)
def _agg_kernel(g_hbm, src_hbm, dst_hbm, zeros_hbm, out_hbm,
                acc_sh, s0, s1, d0, d1, r0, r1, zer_v, gs0, gs1):
    cc = lax.axis_index("c")
    s = lax.axis_index("s")
    wid = cc * NS + s
    base = wid * EPT
    rows = (r0, r1)
    sidx = (s0, s1)
    didx = (d0, d1)
    gsem = (gs0, gs1)

    pltpu.sync_copy(zeros_hbm, zer_v)

    @pl.loop(0, RPT // ZR)
    def _(i):
        pltpu.sync_copy(zer_v, acc_sh.at[pl.ds(s * RPT + i * ZR, ZR)])

    plsc.subcore_barrier()

    def g_desc(b):
        return pltpu.make_async_copy(g_hbm.at[sidx[b]], rows[b], gsem[b])

    for b in range(NB):
        pltpu.sync_copy(src_hbm.at[pl.ds(base + b * CH, CH)], sidx[b])
        pltpu.sync_copy(dst_hbm.at[pl.ds(base + b * CH, CH)], didx[b])
        g_desc(b).start()

    @pl.loop(0, NF // NB - 1)
    def _(i):
        for b in range(NB):
            ch = i * NB + b
            g_desc(b).wait()
            pltpu.sync_copy(rows[b], acc_sh.at[didx[b]], add=True)
            pltpu.sync_copy(src_hbm.at[pl.ds(base + (ch + NB) * CH, CH)],
                            sidx[b])
            pltpu.sync_copy(dst_hbm.at[pl.ds(base + (ch + NB) * CH, CH)],
                            didx[b])
            g_desc(b).start()

    for b in range(NB):
        g_desc(b).wait()
        pltpu.sync_copy(rows[b], acc_sh.at[didx[b]], add=True)

    plsc.subcore_barrier()
    pltpu.sync_copy(acc_sh.at[pl.ds(s * RPT, RPT)],
                    out_hbm.at[pl.ds(cc * NP + s * RPT, RPT)])


BM = 1000


def _mm_body(x_ref, w_ref, d0_ref, d1_ref, g_ref):
    h = jnp.dot(x_ref[...], w_ref[...], preferred_element_type=jnp.float32)
    deg = d0_ref[...] + d1_ref[...] + 1.0
    g_ref[...] = h * lax.rsqrt(deg)


def _mm_call(x, W, deg0, deg1):
    return pl.pallas_call(
        _mm_body,
        grid=(N // BM,),
        in_specs=[
            pl.BlockSpec((BM, D), lambda i: (i, 0)),
            pl.BlockSpec((D, D), lambda i: (0, 0)),
            pl.BlockSpec((BM, 1), lambda i: (i, 0)),
            pl.BlockSpec((BM, 1), lambda i: (i, 0)),
        ],
        out_specs=pl.BlockSpec((BM, D), lambda i: (i, 0)),
        out_shape=jax.ShapeDtypeStruct((N, D), jnp.float32),
    )(x, W, deg0, deg1)


def _fin_body(s_ref, g_ref, d0_ref, d1_ref, x_ref, b_ref, gam_ref, bet_ref,
              o_ref):
    deg = d0_ref[...] + d1_ref[...] + 1.0
    dinv = lax.rsqrt(deg)
    agg = (s_ref[0] + s_ref[1] + g_ref[...]) * dinv + b_ref[...]
    mu = jnp.mean(agg, axis=-1, keepdims=True)
    xc = agg - mu
    var = jnp.mean(xc * xc, axis=-1, keepdims=True)
    y = xc * lax.rsqrt(var + 1e-5) * gam_ref[...] + bet_ref[...]
    o_ref[...] = jnp.maximum(y, 0.0) + x_ref[...]


def _fin_call(S3, g, deg0, deg1, x, b, gamma, beta):
    return pl.pallas_call(
        _fin_body,
        grid=(N // BM,),
        in_specs=[
            pl.BlockSpec((2, BM, D), lambda i: (0, i, 0)),
            pl.BlockSpec((BM, D), lambda i: (i, 0)),
            pl.BlockSpec((BM, 1), lambda i: (i, 0)),
            pl.BlockSpec((BM, 1), lambda i: (i, 0)),
            pl.BlockSpec((BM, D), lambda i: (i, 0)),
            pl.BlockSpec((1, D), lambda i: (0, 0)),
            pl.BlockSpec((1, D), lambda i: (0, 0)),
            pl.BlockSpec((1, D), lambda i: (0, 0)),
        ],
        out_specs=pl.BlockSpec((BM, D), lambda i: (i, 0)),
        out_shape=jax.ShapeDtypeStruct((N, D), jnp.float32),
    )(S3, g, deg0, deg1, x, b, gamma, beta)


def kernel(x, edge_index, W, b, gamma, beta):
    ei = edge_index.astype(jnp.int32)
    src = jnp.concatenate([ei[0], jnp.zeros((EP - E,), jnp.int32)])
    dst = jnp.concatenate([ei[1], jnp.full((EP - E,), NP - 1, jnp.int32)])
    zerD = jnp.zeros((ZR, D), jnp.float32)

    degp = _deg_kernel(dst)
    deg0 = degp[:NP].reshape(NP, 1)
    deg1 = degp[NP:].reshape(NP, 1)
    g = _mm_call(x, W, deg0, deg1)
    S = _agg_kernel(g, src, dst, zerD)
    S3 = S.reshape(NC, NP, D)
    return _fin_call(S3, g, deg0, deg1, x,
                     b.reshape(1, D), gamma.reshape(1, D), beta.reshape(1, D))


# speedup vs baseline: 1.0121x; 1.0121x over previous
import dataclasses
import functools

import jax
import jax.numpy as jnp
from jax import lax
from jax.experimental import pallas as pl
from jax.experimental.pallas import tpu as pltpu
from jax.experimental.pallas import tpu_sc as plsc

N = 10000
E = 320000
D = 128

NC = 2
NS = 16
NW = NC * NS
CH = 128
NF = 80
EPT = NF * CH
EP = NW * EPT
NB = 2
NP = 10240
RPT = NP // NS
ZR = 32

_mesh = plsc.VectorSubcoreMesh(core_axis_name="c", subcore_axis_name="s")

_cp = pltpu.CompilerParams()
if "needs_layout_passes" in pltpu.CompilerParams.__dataclass_fields__:
    _cp = dataclasses.replace(_cp, needs_layout_passes=False)


@functools.partial(
    pl.kernel,
    out_type=jax.ShapeDtypeStruct((NC * NP,), jnp.float32),
    mesh=_mesh,
    scratch_types=[
        pltpu.VMEM((NP,), jnp.float32),
        pltpu.VMEM((EPT,), jnp.int32),
        pltpu.VMEM_SHARED((NS, NP), jnp.float32),
        pltpu.VMEM((RPT,), jnp.float32),
        pltpu.VMEM((RPT,), jnp.float32),
    ],
    compiler_params=_cp,
)
def _deg_kernel(dst_hbm, out_hbm, hist_v, idx_v, shared, tmp_v, acc_v):
    c = lax.axis_index("c")
    s = lax.axis_index("s")
    wid = c * NS + s
    base = wid * EPT
    pltpu.sync_copy(dst_hbm.at[pl.ds(base, EPT)], idx_v)

    zero16 = jnp.zeros((16,), jnp.float32)
    ones16 = jnp.ones((16,), jnp.float32)

    @pl.loop(0, NP // 16)
    def _(i):
        hist_v[pl.ds(i * 16, 16)] = zero16

    @pl.loop(0, EPT // 16)
    def _(j):
        idx16 = idx_v[pl.ds(j * 16, 16)]
        plsc.addupdate_scatter(hist_v, [idx16], ones16)

    pltpu.sync_copy(hist_v, shared.at[s])
    plsc.subcore_barrier()

    pltpu.sync_copy(shared.at[0, pl.ds(s * RPT, RPT)], acc_v)

    @pl.loop(1, NS)
    def _(r):
        pltpu.sync_copy(shared.at[r, pl.ds(s * RPT, RPT)], tmp_v)

        @pl.loop(0, RPT // 16)
        def _(i):
            sl = pl.ds(i * 16, 16)
            acc_v[sl] = acc_v[sl] + tmp_v[sl]

    pltpu.sync_copy(acc_v, out_hbm.at[pl.ds(c * NP + s * RPT, RPT)])


@functools.partial(
    pl.kernel,
    out_type=jax.ShapeDtypeStruct((NC * NP, D), jnp.float32),
    mesh=_mesh,
    scratch_types=[
        pltpu.VMEM_SHARED((NP, D), jnp.float32),
        pltpu.VMEM((CH,), jnp.int32),
        pltpu.VMEM((CH,), jnp.int32),
        pltpu.VMEM((CH,), jnp.int32),
        pltpu.VMEM((CH,), jnp.int32),
        pltpu.VMEM((CH, D), jnp.float32),
        pltpu.VMEM((CH, D), jnp.float32),
        pltpu.VMEM((ZR, D), jnp.float32),
        pltpu.SemaphoreType.DMA,
        pltpu.SemaphoreType.DMA,
    ],
)
def _agg_kernel(g_hbm, src_hbm, dst_hbm, zeros_hbm, out_hbm,
                acc_sh, s0, s1, d0, d1, r0, r1, zer_v, gs0, gs1):
    cc = lax.axis_index("c")
    s = lax.axis_index("s")
    wid = cc * NS + s
    base = wid * EPT
    rows = (r0, r1)
    sidx = (s0, s1)
    didx = (d0, d1)
    gsem = (gs0, gs1)

    pltpu.sync_copy(zeros_hbm, zer_v)

    @pl.loop(0, RPT // ZR)
    def _(i):
        pltpu.sync_copy(zer_v, acc_sh.at[pl.ds(s * RPT + i * ZR, ZR)])

    plsc.subcore_barrier()

    def g_desc(b):
        return pltpu.make_async_copy(g_hbm.at[sidx[b]], rows[b], gsem[b])

    for b in range(NB):
        pltpu.sync_copy(src_hbm.at[pl.ds(base + b * CH, CH)], sidx[b])
        pltpu.sync_copy(dst_hbm.at[pl.ds(base + b * CH, CH)], didx[b])
        g_desc(b).start()

    @pl.loop(0, NF // NB - 1)
    def _(i):
        for b in range(NB):
            ch = i * NB + b
            g_desc(b).wait()
            pltpu.sync_copy(rows[b], acc_sh.at[didx[b]], add=True)
            pltpu.sync_copy(src_hbm.at[pl.ds(base + (ch + NB) * CH, CH)],
                            sidx[b])
            pltpu.sync_copy(dst_hbm.at[pl.ds(base + (ch + NB) * CH, CH)],
                            didx[b])
            g_desc(b).start()

    for b in range(NB):
        g_desc(b).wait()
        pltpu.sync_copy(rows[b], acc_sh.at[didx[b]], add=True)

    plsc.subcore_barrier()
    pltpu.sync_copy(acc_sh.at[pl.ds(s * RPT, RPT)],
                    out_hbm.at[pl.ds(cc * NP + s * RPT, RPT)])


BM = 1000


def _mm_body(x_ref, w_ref, d0_ref, d1_ref, g_ref):
    h = jnp.dot(x_ref[...], w_ref[...], preferred_element_type=jnp.float32)
    deg = d0_ref[...] + d1_ref[...] + 1.0
    g_ref[...] = h * lax.rsqrt(deg)


def _mm_call(x, W, deg0, deg1):
    return pl.pallas_call(
        _mm_body,
        grid=(N // BM,),
        in_specs=[
            pl.BlockSpec((BM, D), lambda i: (i, 0)),
            pl.BlockSpec((D, D), lambda i: (0, 0)),
            pl.BlockSpec((BM, 1), lambda i: (i, 0)),
            pl.BlockSpec((BM, 1), lambda i: (i, 0)),
        ],
        out_specs=pl.BlockSpec((BM, D), lambda i: (i, 0)),
        out_shape=jax.ShapeDtypeStruct((N, D), jnp.float32),
    )(x, W, deg0, deg1)


def _fin_body(s_ref, g_ref, d0_ref, d1_ref, x_ref, b_ref, gam_ref, bet_ref,
              o_ref):
    deg = d0_ref[...] + d1_ref[...] + 1.0
    dinv = lax.rsqrt(deg)
    agg = (s_ref[0] + s_ref[1] + g_ref[...]) * dinv + b_ref[...]
    mu = jnp.mean(agg, axis=-1, keepdims=True)
    xc = agg - mu
    var = jnp.mean(xc * xc, axis=-1, keepdims=True)
    y = xc * lax.rsqrt(var + 1e-5) * gam_ref[...] + bet_ref[...]
    o_ref[...] = jnp.maximum(y, 0.0) + x_ref[...]


def _fin_call(S3, g, deg0, deg1, x, b, gamma, beta):
    return pl.pallas_call(
        _fin_body,
        grid=(N // BM,),
        in_specs=[
            pl.BlockSpec((2, BM, D), lambda i: (0, i, 0)),
            pl.BlockSpec((BM, D), lambda i: (i, 0)),
            pl.BlockSpec((BM, 1), lambda i: (i, 0)),
            pl.BlockSpec((BM, 1), lambda i: (i, 0)),
            pl.BlockSpec((BM, D), lambda i: (i, 0)),
            pl.BlockSpec((1, D), lambda i: (0, 0)),
            pl.BlockSpec((1, D), lambda i: (0, 0)),
            pl.BlockSpec((1, D), lambda i: (0, 0)),
        ],
        out_specs=pl.BlockSpec((BM, D), lambda i: (i, 0)),
        out_shape=jax.ShapeDtypeStruct((N, D), jnp.float32),
    )(S3, g, deg0, deg1, x, b, gamma, beta)


def kernel(x, edge_index, W, b, gamma, beta):
    ei = edge_index.astype(jnp.int32)
    src = jnp.concatenate([ei[0], jnp.zeros((EP - E,), jnp.int32)])
    pad_dst = N + jnp.arange(EP - E, dtype=jnp.int32) % (NP - N)
    dst = jnp.concatenate([ei[1], pad_dst])
    zerD = jnp.zeros((ZR, D), jnp.float32)

    degp = _deg_kernel(dst)
    deg0 = degp[:NP].reshape(NP, 1)
    deg1 = degp[NP:].reshape(NP, 1)
    g = _mm_call(x, W, deg0, deg1)
    S = _agg_kernel(g, src, dst, zerD)
    S3 = S.reshape(NC, NP, D)
    return _fin_call(S3, g, deg0, deg1, x,
                     b.reshape(1, D), gamma.reshape(1, D), beta.reshape(1, D))


# speedup vs baseline: 2.5629x; 2.5323x over previous
import dataclasses
import functools

import jax
import jax.numpy as jnp
from jax import lax
from jax.experimental import pallas as pl
from jax.experimental.pallas import tpu as pltpu
from jax.experimental.pallas import tpu_sc as plsc

N = 10000
E = 320000
D = 128

NC = 2
NS = 16
NW = NC * NS
CH = 128
NF = 80
EPT = NF * CH
EP = NW * EPT
NB = 2
NP = 10240
RPT = NP // NS
ZR = 32

_mesh = plsc.VectorSubcoreMesh(core_axis_name="c", subcore_axis_name="s")

_cp = pltpu.CompilerParams()
if "needs_layout_passes" in pltpu.CompilerParams.__dataclass_fields__:
    _cp = dataclasses.replace(_cp, needs_layout_passes=False)


@functools.partial(
    pl.kernel,
    out_type=jax.ShapeDtypeStruct((NC * NP,), jnp.float32),
    mesh=_mesh,
    scratch_types=[
        pltpu.VMEM((NP,), jnp.float32),
        pltpu.VMEM((EPT,), jnp.int32),
        pltpu.VMEM_SHARED((NS, NP), jnp.float32),
        pltpu.VMEM((RPT,), jnp.float32),
        pltpu.VMEM((RPT,), jnp.float32),
    ],
    compiler_params=_cp,
)
def _deg_kernel(dst_hbm, out_hbm, hist_v, idx_v, shared, tmp_v, acc_v):
    c = lax.axis_index("c")
    s = lax.axis_index("s")
    wid = c * NS + s
    base = wid * EPT
    pltpu.sync_copy(dst_hbm.at[pl.ds(base, EPT)], idx_v)

    zero16 = jnp.zeros((16,), jnp.float32)
    ones16 = jnp.ones((16,), jnp.float32)

    @pl.loop(0, NP // 16)
    def _(i):
        hist_v[pl.ds(i * 16, 16)] = zero16

    @pl.loop(0, EPT // 16)
    def _(j):
        idx16 = idx_v[pl.ds(j * 16, 16)]
        plsc.addupdate_scatter(hist_v, [idx16], ones16)

    pltpu.sync_copy(hist_v, shared.at[s])
    plsc.subcore_barrier()

    pltpu.sync_copy(shared.at[0, pl.ds(s * RPT, RPT)], acc_v)

    @pl.loop(1, NS)
    def _(r):
        pltpu.sync_copy(shared.at[r, pl.ds(s * RPT, RPT)], tmp_v)

        @pl.loop(0, RPT // 16)
        def _(i):
            sl = pl.ds(i * 16, 16)
            acc_v[sl] = acc_v[sl] + tmp_v[sl]

    pltpu.sync_copy(acc_v, out_hbm.at[pl.ds(c * NP + s * RPT, RPT)])


@functools.partial(
    pl.kernel,
    out_type=jax.ShapeDtypeStruct((NC * NP, D), jnp.float32),
    mesh=_mesh,
    scratch_types=[
        pltpu.VMEM_SHARED((NP, D), jnp.float32),
        pltpu.VMEM((CH,), jnp.int32),
        pltpu.VMEM((CH,), jnp.int32),
        pltpu.VMEM((CH,), jnp.int32),
        pltpu.VMEM((CH,), jnp.int32),
        pltpu.VMEM((CH, D), jnp.float32),
        pltpu.VMEM((CH, D), jnp.float32),
        pltpu.VMEM((ZR, D), jnp.float32),
        pltpu.SemaphoreType.DMA,
        pltpu.SemaphoreType.DMA,
    ],
)
def _agg_kernel(g_hbm, src_hbm, dst_hbm, zeros_hbm, out_hbm,
                acc_sh, s0, s1, d0, d1, r0, r1, zer_v, gs0, gs1):
    cc = lax.axis_index("c")
    s = lax.axis_index("s")
    wid = cc * NS + s
    base = wid * EPT
    rows = (r0, r1)
    sidx = (s0, s1)
    didx = (d0, d1)
    gsem = (gs0, gs1)

    pltpu.sync_copy(zeros_hbm, zer_v)

    @pl.loop(0, RPT // ZR)
    def _(i):
        pltpu.sync_copy(zer_v, acc_sh.at[pl.ds(s * RPT + i * ZR, ZR)])

    plsc.subcore_barrier()

    def g_desc(b):
        return pltpu.make_async_copy(g_hbm.at[sidx[b]], rows[b], gsem[b])

    for b in range(NB):
        pltpu.sync_copy(src_hbm.at[pl.ds(base + b * CH, CH)], sidx[b])
        pltpu.sync_copy(dst_hbm.at[pl.ds(base + b * CH, CH)], didx[b])
        g_desc(b).start()

    @pl.loop(0, NF // NB - 1)
    def _(i):
        for b in range(NB):
            ch = i * NB + b
            g_desc(b).wait()
            pltpu.sync_copy(rows[b], acc_sh.at[didx[b]], add=True)
            pltpu.sync_copy(src_hbm.at[pl.ds(base + (ch + NB) * CH, CH)],
                            sidx[b])
            pltpu.sync_copy(dst_hbm.at[pl.ds(base + (ch + NB) * CH, CH)],
                            didx[b])
            g_desc(b).start()

    for b in range(NB):
        g_desc(b).wait()
        pltpu.sync_copy(rows[b], acc_sh.at[didx[b]], add=True)

    plsc.subcore_barrier()
    pltpu.sync_copy(acc_sh.at[pl.ds(s * RPT, RPT)],
                    out_hbm.at[pl.ds(cc * NP + s * RPT, RPT)])


BM = 1000


def _mm_body(x_ref, w_ref, d0_ref, d1_ref, g_ref):
    h = jnp.dot(x_ref[...], w_ref[...], preferred_element_type=jnp.float32)
    deg = d0_ref[...] + d1_ref[...] + 1.0
    g_ref[...] = h * lax.rsqrt(deg)


def _mm_call(x, W, deg0, deg1):
    return pl.pallas_call(
        _mm_body,
        grid=(N // BM,),
        in_specs=[
            pl.BlockSpec((BM, D), lambda i: (i, 0)),
            pl.BlockSpec((D, D), lambda i: (0, 0)),
            pl.BlockSpec((BM, 1), lambda i: (i, 0)),
            pl.BlockSpec((BM, 1), lambda i: (i, 0)),
        ],
        out_specs=pl.BlockSpec((BM, D), lambda i: (i, 0)),
        out_shape=jax.ShapeDtypeStruct((N, D), jnp.float32),
    )(x, W, deg0, deg1)


def _fin_body(s_ref, g_ref, d0_ref, d1_ref, x_ref, b_ref, gam_ref, bet_ref,
              o_ref):
    deg = d0_ref[...] + d1_ref[...] + 1.0
    dinv = lax.rsqrt(deg)
    agg = (s_ref[0] + s_ref[1] + g_ref[...]) * dinv + b_ref[...]
    mu = jnp.mean(agg, axis=-1, keepdims=True)
    xc = agg - mu
    var = jnp.mean(xc * xc, axis=-1, keepdims=True)
    y = xc * lax.rsqrt(var + 1e-5) * gam_ref[...] + bet_ref[...]
    o_ref[...] = jnp.maximum(y, 0.0) + x_ref[...]


def _fin_call(S3, g, deg0, deg1, x, b, gamma, beta):
    return pl.pallas_call(
        _fin_body,
        grid=(N // BM,),
        in_specs=[
            pl.BlockSpec((2, BM, D), lambda i: (0, i, 0)),
            pl.BlockSpec((BM, D), lambda i: (i, 0)),
            pl.BlockSpec((BM, 1), lambda i: (i, 0)),
            pl.BlockSpec((BM, 1), lambda i: (i, 0)),
            pl.BlockSpec((BM, D), lambda i: (i, 0)),
            pl.BlockSpec((1, D), lambda i: (0, 0)),
            pl.BlockSpec((1, D), lambda i: (0, 0)),
            pl.BlockSpec((1, D), lambda i: (0, 0)),
        ],
        out_specs=pl.BlockSpec((BM, D), lambda i: (i, 0)),
        out_shape=jax.ShapeDtypeStruct((N, D), jnp.float32),
    )(S3, g, deg0, deg1, x, b, gamma, beta)


def kernel(x, edge_index, W, b, gamma, beta):
    ei = edge_index.astype(jnp.int32)
    ppt = (EP - E) // NW
    lane = jnp.arange(ppt, dtype=jnp.int32)
    src_pad = jnp.broadcast_to(lane[None, :], (NW, ppt))
    dst_pad = jnp.broadcast_to((N + lane)[None, :], (NW, ppt))
    src = jnp.concatenate([ei[0].reshape(NW, E // NW), src_pad], 1).reshape(-1)
    dst = jnp.concatenate([ei[1].reshape(NW, E // NW), dst_pad], 1).reshape(-1)
    zerD = jnp.zeros((ZR, D), jnp.float32)

    degp = _deg_kernel(dst)
    deg0 = degp[:NP].reshape(NP, 1)
    deg1 = degp[NP:].reshape(NP, 1)
    g = _mm_call(x, W, deg0, deg1)
    S = _agg_kernel(g, src, dst, zerD)
    S3 = S.reshape(NC, NP, D)
    return _fin_call(S3, g, deg0, deg1, x,
                     b.reshape(1, D), gamma.reshape(1, D), beta.reshape(1, D))
